# Initial kernel scaffold; baseline (speedup 1.0000x reference)
import jax
import jax.numpy as jnp
from jax.experimental import pallas as pl


def kernel(x, W1, b1, W2, b2, gn1_w, gn1_b, gn2_w, gn2_b):
    raise NotImplementedError("write your pallas kernel here")


# speedup vs baseline: 26.2984x; 26.2984x over previous
import functools

import jax
import jax.numpy as jnp
import numpy as np
from jax.experimental import pallas as pl
from jax.experimental.pallas import tpu as pltpu

_B, _C, _H, _W = 8, 96, 32, 32
_N = _H * _W
_K = 9
_GROUPS = 4
_GSIZE = _C // _GROUPS


def _gcn_body(x_nc_ref, x_cn_ref, W1_ref, b1_ref, W2_ref, b2_ref,
              gn1w_ref, gn1b_ref, gn2w_ref, gn2b_ref, M_ref,
              out_ref, sim_ref, A_ref):
    x_nc = x_nc_ref[0]
    x_cn = x_cn_ref[0]

    rs = jnp.sum(x_nc * x_nc, axis=1, keepdims=True)
    inv_r = 1.0 / jnp.maximum(jnp.sqrt(rs), 1e-12)
    xn = x_nc * inv_r
    cs = jnp.sum(x_cn * x_cn, axis=0, keepdims=True)
    inv_c = 1.0 / jnp.maximum(jnp.sqrt(cs), 1e-12)
    xnT = x_cn * inv_c

    sim_ref[...] = jnp.dot(xn, xnT, preferred_element_type=jnp.float32)

    A_ref[...] = jnp.zeros((_N, _N), jnp.float32)
    lane = jax.lax.broadcasted_iota(jnp.int32, (_N, _N), 1)
    deg = jnp.zeros((_N, 1), jnp.float32)
    for _ in range(_K):
        s = sim_ref[...]
        m = jnp.max(s, axis=1, keepdims=True)
        cand = jnp.where(s == m, lane, _N)
        sel = jnp.min(cand, axis=1, keepdims=True)
        onehot = lane == sel
        A_ref[...] += jnp.where(onehot, s, 0.0)
        deg = deg + m
        sim_ref[...] = jnp.where(onehot, -3.0, s)
    inv_deg = 1.0 / (deg + 1e-6)

    M = M_ref[...]

    def layer(feat, Wt, bias, gnw, gnb):
        xt = jnp.dot(feat, Wt, preferred_element_type=jnp.float32)
        agg = jnp.dot(A_ref[...], xt,
                      preferred_element_type=jnp.float32) * inv_deg + bias
        csum = jnp.sum(agg, axis=0, keepdims=True)
        csq = jnp.sum(agg * agg, axis=0, keepdims=True)
        mean = jnp.dot(csum, M, preferred_element_type=jnp.float32)
        ex2 = jnp.dot(csq, M, preferred_element_type=jnp.float32)
        var = ex2 - mean * mean
        hn = (agg - mean) * jax.lax.rsqrt(var + 1e-5)
        hn = hn * gnw + gnb
        return hn * (1.0 / (1.0 + jnp.exp(-hn)))

    h = layer(xn, W1_ref[...], b1_ref[...], gn1w_ref[...], gn1b_ref[...])
    out = layer(h, W2_ref[...], b2_ref[...], gn2w_ref[...], gn2b_ref[...])
    out_ref[0] = out


@jax.jit
def _run(x, W1, b1, W2, b2, gn1w, gn1b, gn2w, gn2b):
    x_cn = x.reshape(_B, _C, _N)
    x_nc = x_cn.transpose(0, 2, 1)
    g = np.arange(_C) // _GSIZE
    M = jnp.asarray((g[:, None] == g[None, :]).astype(np.float32)
                    / (_N * _GSIZE))

    full = lambda *shape: pl.BlockSpec(shape, lambda b: (0,) * len(shape))
    out = pl.pallas_call(
        _gcn_body,
        grid=(_B,),
        in_specs=[
            pl.BlockSpec((1, _N, _C), lambda b: (b, 0, 0)),
            pl.BlockSpec((1, _C, _N), lambda b: (b, 0, 0)),
            full(_C, _C), full(1, _C), full(_C, _C), full(1, _C),
            full(1, _C), full(1, _C), full(1, _C), full(1, _C),
            full(_C, _C),
        ],
        out_specs=pl.BlockSpec((1, _N, _C), lambda b: (b, 0, 0)),
        out_shape=jax.ShapeDtypeStruct((_B, _N, _C), jnp.float32),
        scratch_shapes=[pltpu.VMEM((_N, _N), jnp.float32),
                        pltpu.VMEM((_N, _N), jnp.float32)],
    )(x_nc, x_cn, W1, b1.reshape(1, _C), W2, b2.reshape(1, _C),
      gn1w.reshape(1, _C), gn1b.reshape(1, _C),
      gn2w.reshape(1, _C), gn2b.reshape(1, _C), M)
    return out.transpose(0, 2, 1).reshape(_B, _C, _H, _W)


def kernel(x, W1, b1, W2, b2, gn1_w, gn1_b, gn2_w, gn2_b):
    return _run(x, W1, b1, W2, b2, gn1_w, gn1_b, gn2_w, gn2_b)
